# Initial kernel scaffold; baseline (speedup 1.0000x reference)
import functools

import jax
import jax.numpy as jnp
from jax import lax
from jax.experimental import pallas as pl
from jax.experimental.pallas import tpu as pltpu
from jax.experimental.pallas import tpu_sc as plsc

_B, _C, _N = 16, 512, 2048
_MAX_H, _MAX_W = 70, 40
_GRID = 300
_HW = _MAX_H * _MAX_W
_ZSLOT = _N
_NC, _NS = 2, 16
_CPT = _C // _NC


def _tc_pix_body(feat_ref, ys_ref, xs_ref, pix_ref):
    yb = ys_ref[0]
    xb = xs_ref[0]
    valid = yb > -1
    min_y = jnp.min(jnp.where(valid, yb, _GRID))
    max_y = jnp.max(jnp.where(valid, yb, -1))
    min_x = jnp.min(jnp.where(valid, xb, _GRID))
    max_x = jnp.max(jnp.where(valid, xb, -1))
    h0 = max_y - min_y + 1
    w0 = max_x - min_x + 1
    swap = w0 > h0
    height = jnp.where(swap, w0, h0)
    width = jnp.where(swap, h0, w0)
    h_dif = height - _MAX_H
    w_dif = width - _MAX_W
    cut_top = jnp.where(h_dif > 0, (h_dif + 1) // 2, 0)
    pad_top = jnp.where(h_dif > 0, 0, (-h_dif + 1) // 2)
    cut_left = jnp.where(w_dif > 0, (w_dif + 1) // 2, 0)
    pad_right = jnp.where(w_dif > 0, 0, (-w_dif + 1) // 2)
    ry = yb - min_y
    rx = xb - min_x
    row = jnp.where(swap, rx, ry)
    col = jnp.where(swap, ry, rx)
    r = row - cut_top + pad_top
    c = col - cut_left + pad_right
    inb = (r >= 0) & (r < _MAX_H) & (c >= 0) & (c < _MAX_W)
    f = feat_ref[0]
    real = jnp.min(jnp.where(f != -1.0, 1, 0).astype(jnp.int32),
                   axis=0, keepdims=True)
    pix = jnp.where(valid & inb & (real > 0), r * _MAX_W + c, -1)
    pix_ref[0] = pix.astype(jnp.int32)


def _tc_pix(features, ys3, xs3):
    return pl.pallas_call(
        _tc_pix_body,
        grid=(_B,),
        in_specs=[
            pl.BlockSpec((1, _C, _N), lambda b: (b, 0, 0)),
            pl.BlockSpec((1, 1, _N), lambda b: (b, 0, 0)),
            pl.BlockSpec((1, 1, _N), lambda b: (b, 0, 0)),
        ],
        out_specs=pl.BlockSpec((1, 1, _N), lambda b: (b, 0, 0)),
        out_shape=jax.ShapeDtypeStruct((_B, 1, _N), jnp.int32),
    )(features, ys3, xs3)


@functools.partial(
    pl.kernel,
    mesh=plsc.VectorSubcoreMesh(core_axis_name="c", subcore_axis_name="s"),
    out_type=jax.ShapeDtypeStruct((_B * _C, _HW), jnp.float32),
    scratch_types=[
        pltpu.VMEM((_N,), jnp.int32),
        pltpu.VMEM((_HW,), jnp.int32),
        pltpu.VMEM((_N + 16,), jnp.float32),
        pltpu.VMEM((_HW,), jnp.float32),
    ],
)
def _sc_gather(feat_hbm, pix_hbm, out_hbm, pix_v, imap_v, tab_v, out_v):
    cid = lax.axis_index("c")
    sid = lax.axis_index("s")
    b = sid
    c0 = cid * _CPT
    pltpu.sync_copy(pix_hbm.at[b], pix_v)
    tab_v[pl.ds(_N, 16)] = jnp.zeros((16,), jnp.float32)
    zvec = jnp.full((16,), _ZSLOT, jnp.int32)
    for k in range(_HW // 16):
        imap_v[pl.ds(k * 16, 16)] = zvec
    iota16 = lax.iota(jnp.int32, 16)
    for j in range(_N // 16):
        idx = pix_v[pl.ds(j * 16, 16)]
        m = idx >= 0
        plsc.store_scatter(imap_v, [jnp.maximum(idx, 0)], iota16 + (j * 16), m)

    def chan_body(c, carry):
        ch = c0 + c
        pltpu.sync_copy(feat_hbm.at[b, ch], tab_v.at[pl.ds(0, _N)])
        for k in range(_HW // 16):
            im = imap_v[pl.ds(k * 16, 16)]
            out_v[pl.ds(k * 16, 16)] = plsc.load_gather(tab_v, [im])
        pltpu.sync_copy(out_v, out_hbm.at[b * _C + ch])
        return carry

    lax.fori_loop(0, _CPT, chan_body, 0)


def kernel(features, ys, xs):
    ys3 = ys.reshape(_B, 1, _N)
    xs3 = xs.reshape(_B, 1, _N)
    pix = _tc_pix(features, ys3, xs3)
    out = _sc_gather(features, pix.reshape(_B, _N))
    return out.reshape(_B, _C, _MAX_H, _MAX_W)


# speedup vs baseline: 16.4252x; 16.4252x over previous
import functools

import jax
import jax.numpy as jnp
from jax import lax
from jax.experimental import pallas as pl
from jax.experimental.pallas import tpu as pltpu
from jax.experimental.pallas import tpu_sc as plsc

_B, _C, _N = 16, 512, 2048
_MAX_H, _MAX_W = 70, 40
_GRID = 300
_HW = _MAX_H * _MAX_W
_ZSLOT = _N
_NC, _NS = 2, 16
_CPT = _C // _NC


def _tc_pix_body(feat_ref, ys_ref, xs_ref, pix_ref):
    yb = ys_ref[0]
    xb = xs_ref[0]
    valid = yb > -1
    min_y = jnp.min(jnp.where(valid, yb, _GRID))
    max_y = jnp.max(jnp.where(valid, yb, -1))
    min_x = jnp.min(jnp.where(valid, xb, _GRID))
    max_x = jnp.max(jnp.where(valid, xb, -1))
    h0 = max_y - min_y + 1
    w0 = max_x - min_x + 1
    swap = w0 > h0
    height = jnp.where(swap, w0, h0)
    width = jnp.where(swap, h0, w0)
    h_dif = height - _MAX_H
    w_dif = width - _MAX_W
    cut_top = jnp.where(h_dif > 0, (h_dif + 1) // 2, 0)
    pad_top = jnp.where(h_dif > 0, 0, (-h_dif + 1) // 2)
    cut_left = jnp.where(w_dif > 0, (w_dif + 1) // 2, 0)
    pad_right = jnp.where(w_dif > 0, 0, (-w_dif + 1) // 2)
    ry = yb - min_y
    rx = xb - min_x
    row = jnp.where(swap, rx, ry)
    col = jnp.where(swap, ry, rx)
    r = row - cut_top + pad_top
    c = col - cut_left + pad_right
    inb = (r >= 0) & (r < _MAX_H) & (c >= 0) & (c < _MAX_W)
    f = feat_ref[0]
    real = jnp.min(jnp.where(f != -1.0, 1, 0).astype(jnp.int32),
                   axis=0, keepdims=True)
    pix = jnp.where(valid & inb & (real > 0), r * _MAX_W + c, -1)
    pix_ref[0] = pix.astype(jnp.int32)


def _tc_pix(features, ys3, xs3):
    return pl.pallas_call(
        _tc_pix_body,
        grid=(_B,),
        in_specs=[
            pl.BlockSpec((1, _C, _N), lambda b: (b, 0, 0)),
            pl.BlockSpec((1, 1, _N), lambda b: (b, 0, 0)),
            pl.BlockSpec((1, 1, _N), lambda b: (b, 0, 0)),
        ],
        out_specs=pl.BlockSpec((1, 1, _N), lambda b: (b, 0, 0)),
        out_shape=jax.ShapeDtypeStruct((_B, 1, _N), jnp.int32),
    )(features, ys3, xs3)


@functools.lru_cache(maxsize=None)
def _sc_gather_fn():
    return functools.partial(
        pl.kernel,
        mesh=plsc.VectorSubcoreMesh(core_axis_name="c", subcore_axis_name="s"),
        compiler_params=pltpu.CompilerParams(needs_layout_passes=False),
        out_type=jax.ShapeDtypeStruct((_B * _C, _HW), jnp.float32),
        scratch_types=[
            pltpu.VMEM((_N,), jnp.int32),
            pltpu.VMEM((_HW,), jnp.int32),
            pltpu.VMEM((_N + 16,), jnp.float32),
            pltpu.VMEM((_HW,), jnp.float32),
        ],
    )(_sc_gather_body)


def _sc_gather_body(feat_hbm, pix_hbm, out_hbm, pix_v, imap_v, tab_v, out_v):
    cid = lax.axis_index("c")
    sid = lax.axis_index("s")
    b = sid
    c0 = cid * _CPT
    pltpu.sync_copy(pix_hbm.at[b], pix_v)
    tab_v[pl.ds(_N, 16)] = jnp.zeros((16,), jnp.float32)
    zvec = jnp.full((16,), _ZSLOT, jnp.int32)
    for k in range(_HW // 16):
        imap_v[pl.ds(k * 16, 16)] = zvec
    iota16 = lax.iota(jnp.int32, 16)
    for j in range(_N // 16):
        idx = pix_v[pl.ds(j * 16, 16)]
        m = idx >= 0
        plsc.store_scatter(imap_v, [jnp.maximum(idx, 0)], iota16 + (j * 16), mask=m)

    def chan_body(c, carry):
        ch = c0 + c
        pltpu.sync_copy(feat_hbm.at[b, ch], tab_v.at[pl.ds(0, _N)])
        for k in range(_HW // 16):
            im = imap_v[pl.ds(k * 16, 16)]
            out_v[pl.ds(k * 16, 16)] = plsc.load_gather(tab_v, [im])
        pltpu.sync_copy(out_v, out_hbm.at[b * _C + ch])
        return carry

    lax.fori_loop(0, _CPT, chan_body, 0)


def kernel(features, ys, xs):
    ys3 = ys.reshape(_B, 1, _N)
    xs3 = xs.reshape(_B, 1, _N)
    pix = _tc_pix(features, ys3, xs3)
    out = _sc_gather_fn()(features, pix.reshape(_B, _N))
    return out.reshape(_B, _C, _MAX_H, _MAX_W)
